# baseline (speedup 1.0000x reference)
import jax
import jax.numpy as jnp
from jax import lax
from jax.experimental import pallas as pl
from jax.experimental.pallas import tpu as pltpu
from jax.experimental.pallas import tpu_sc as plsc

N = 10000
E = 320000
F = 128
NCLS = 40
CP = 48

NC, NS, LANES = 2, 16, 16
NW = NC * NS
EPW = E // NW
CHUNK = 100
NCH = EPW // CHUNK
NH = NCH // 2
RPW = N // NS

_mesh = plsc.VectorSubcoreMesh(core_axis_name="c", subcore_axis_name="s")
_sc_params = pltpu.CompilerParams(use_tc_tiling_on_sc=False)


def _make_agg(fw):
    def body(h_hbm, srcb_hbm, dstb_hbm, zrow_hbm, out_hbm,
             isrc, idst, rows0, rows1, acc, gsem0, gsem1, ssem0, ssem1):
        c = lax.axis_index("c")
        s = lax.axis_index("s")
        wid = s * NC + c
        r0 = s * RPW

        pltpu.sync_copy(srcb_hbm.at[wid], isrc)
        pltpu.sync_copy(dstb_hbm.at[wid], idst)
        pltpu.sync_copy(zrow_hbm.at[pl.ds(r0, RPW)], acc.at[pl.ds(r0, RPW)])
        plsc.subcore_barrier()

        pltpu.async_copy(h_hbm.at[isrc.at[0]], rows0, gsem0)

        @pl.loop(0, NH)
        def step(j):
            i0 = 2 * j
            i1 = i0 + 1
            pltpu.make_async_copy(h_hbm.at[isrc.at[i0]], rows0, gsem0).wait()

            @pl.when(j > 0)
            def _():
                pltpu.make_async_copy(rows1, acc.at[idst.at[i0 - 1]],
                                      ssem1).wait()

            pltpu.async_copy(h_hbm.at[isrc.at[i1]], rows1, gsem1)
            pltpu.async_copy(rows0, acc.at[idst.at[i0]], ssem0, add=True)
            pltpu.make_async_copy(h_hbm.at[isrc.at[i1]], rows1, gsem1).wait()
            pltpu.make_async_copy(rows0, acc.at[idst.at[i0]], ssem0).wait()
            pltpu.async_copy(rows1, acc.at[idst.at[i1]], ssem1, add=True)

            @pl.when(j < NH - 1)
            def _():
                pltpu.async_copy(h_hbm.at[isrc.at[i0 + 2]], rows0, gsem0)

        pltpu.make_async_copy(rows1, acc.at[idst.at[NCH - 1]], ssem1).wait()
        plsc.subcore_barrier()
        pltpu.sync_copy(acc.at[pl.ds(r0, RPW)],
                        out_hbm.at[c, pl.ds(r0, RPW)])

    return pl.kernel(
        body,
        out_type=jax.ShapeDtypeStruct((NC, N, fw), jnp.float32),
        mesh=_mesh,
        scratch_types=(
            pltpu.VMEM((NCH, CHUNK), jnp.int32),
            pltpu.VMEM((NCH, CHUNK), jnp.int32),
            pltpu.VMEM((CHUNK, fw), jnp.float32),
            pltpu.VMEM((CHUNK, fw), jnp.float32),
            pltpu.VMEM_SHARED((N, fw), jnp.float32),
            pltpu.SemaphoreType.DMA,
            pltpu.SemaphoreType.DMA,
            pltpu.SemaphoreType.DMA,
            pltpu.SemaphoreType.DMA,
        ),
        compiler_params=_sc_params,
    )


_agg = _make_agg(F)
_agg_cp = _make_agg(CP)


def _deg_body(dstb_hbm, zdeg_hbm, ones_hbm, deg_hbm,
              idst, ones_v, dacc, ssem):
    c = lax.axis_index("c")
    s = lax.axis_index("s")
    wid = s * NC + c
    r0 = s * RPW

    pltpu.sync_copy(dstb_hbm.at[wid], idst)
    pltpu.sync_copy(ones_hbm, ones_v)
    pltpu.sync_copy(zdeg_hbm.at[pl.ds(r0, RPW)], dacc.at[pl.ds(r0, RPW)])
    plsc.subcore_barrier()

    @pl.loop(0, NCH)
    def step(i):
        pltpu.async_copy(ones_v, dacc.at[idst.at[i]], ssem, add=True)

        @pl.when(i >= 8)
        def _():
            pltpu.make_async_copy(ones_v, dacc.at[idst.at[0]], ssem).wait()

    for _ in range(8):
        pltpu.make_async_copy(ones_v, dacc.at[idst.at[0]], ssem).wait()
    plsc.subcore_barrier()
    pltpu.sync_copy(dacc.at[pl.ds(r0, RPW)],
                    deg_hbm.at[c, pl.ds(r0, RPW)])


_deg = pl.kernel(
    _deg_body,
    out_type=jax.ShapeDtypeStruct((NC, N, LANES), jnp.float32),
    mesh=_mesh,
    scratch_types=(
        pltpu.VMEM((NCH, CHUNK), jnp.int32),
        pltpu.VMEM((CHUNK, LANES), jnp.float32),
        pltpu.VMEM_SHARED((N, LANES), jnp.float32),
        pltpu.SemaphoreType.DMA,
    ),
    compiler_params=_sc_params,
)


_RB = 1000


def _dot_t(a, w):
    return lax.dot_general(a, w, (((1,), (1,)), ((), ())),
                           preferred_element_type=jnp.float32)


def _tc_layer_body(h_ref, sums_ref, deg_ref, ws_ref, wn_ref, b_ref,
                   g_ref, be_ref, mu_ref, var_ref, o_ref):
    h = h_ref[...]
    sums = sums_ref[0] + sums_ref[1]
    deg = deg_ref[0, :, 0:1] + deg_ref[1, :, 0:1]
    hn = sums / jnp.maximum(deg, 1.0)
    z = _dot_t(h, ws_ref[...]) + _dot_t(hn, wn_ref[...]) + b_ref[...]
    scale = g_ref[...] * lax.rsqrt(var_ref[...] + 1e-5)
    shift = be_ref[...] - mu_ref[...] * scale
    y = z * scale + shift
    o_ref[...] = jnp.where(y >= 0, y, 0.01 * y)


def _tc_layer2_body(h_ref, sums_ref, deg_ref, ws_ref, wn_ref, b_ref,
                    g_ref, be_ref, mu_ref, var_ref, wn3p_ref,
                    o_ref, p_ref):
    h = h_ref[...]
    sums = sums_ref[0] + sums_ref[1]
    deg = deg_ref[0, :, 0:1] + deg_ref[1, :, 0:1]
    hn = sums / jnp.maximum(deg, 1.0)
    z = _dot_t(h, ws_ref[...]) + _dot_t(hn, wn_ref[...]) + b_ref[...]
    scale = g_ref[...] * lax.rsqrt(var_ref[...] + 1e-5)
    shift = be_ref[...] - mu_ref[...] * scale
    y = z * scale + shift
    h2 = jnp.where(y >= 0, y, 0.01 * y)
    o_ref[...] = h2
    p_ref[...] = _dot_t(h2, wn3p_ref[...])


def _tc_final_body(h_ref, psums_ref, deg_ref, ws3_ref, b3_ref, o_ref):
    h = h_ref[...]
    psums = psums_ref[0] + psums_ref[1]
    deg = deg_ref[0, :, 0:1] + deg_ref[1, :, 0:1]
    pn = psums / jnp.maximum(deg, 1.0)
    o_ref[...] = _dot_t(h, ws3_ref[...]) + b3_ref[...] + pn[:, :NCLS]


def _row_spec(w):
    return pl.BlockSpec((_RB, w), lambda i: (i, 0))


def _full_spec(shape):
    nd = len(shape)
    return pl.BlockSpec(shape, lambda i, _n=nd: (0,) * _n)


def _sums_spec(w):
    return pl.BlockSpec((NC, _RB, w), lambda i: (0, i, 0))


_GRID = N // _RB

_tc_layer = pl.pallas_call(
    _tc_layer_body,
    grid=(_GRID,),
    in_specs=[_row_spec(F), _sums_spec(F), _sums_spec(LANES),
              _full_spec((F, F)), _full_spec((F, F)), _full_spec((1, F)),
              _full_spec((1, F)), _full_spec((1, F)), _full_spec((1, F)),
              _full_spec((1, F))],
    out_specs=_row_spec(F),
    out_shape=jax.ShapeDtypeStruct((N, F), jnp.float32),
)

_tc_layer2 = pl.pallas_call(
    _tc_layer2_body,
    grid=(_GRID,),
    in_specs=[_row_spec(F), _sums_spec(F), _sums_spec(LANES),
              _full_spec((F, F)), _full_spec((F, F)), _full_spec((1, F)),
              _full_spec((1, F)), _full_spec((1, F)), _full_spec((1, F)),
              _full_spec((1, F)), _full_spec((CP, F))],
    out_specs=[_row_spec(F), _row_spec(CP)],
    out_shape=[jax.ShapeDtypeStruct((N, F), jnp.float32),
               jax.ShapeDtypeStruct((N, CP), jnp.float32)],
)

_tc_final = pl.pallas_call(
    _tc_final_body,
    grid=(_GRID,),
    in_specs=[_row_spec(F), _sums_spec(CP), _sums_spec(LANES),
              _full_spec((NCLS, F)), _full_spec((1, NCLS))],
    out_specs=_row_spec(NCLS),
    out_shape=jax.ShapeDtypeStruct((N, NCLS), jnp.float32),
)


def kernel(x, edge_index, W_self1, W_neigh1, b1, W_self2, W_neigh2, b2,
           W_self3, W_neigh3, b3, bn_gamma, bn_beta, bn_mean, bn_var):
    zrow = jnp.zeros((N, F), jnp.float32)
    zdeg = jnp.zeros((N, LANES), jnp.float32)
    ones = jnp.ones((CHUNK, LANES), jnp.float32)

    r1 = lambda v: v.reshape(1, -1)
    g, be, mu, var = r1(bn_gamma), r1(bn_beta), r1(bn_mean), r1(bn_var)

    wn3p = jnp.pad(W_neigh3, ((0, CP - W_neigh3.shape[0]), (0, 0)))

    srcb = edge_index[0].reshape(NW, NCH, CHUNK)
    dstb = edge_index[1].reshape(NW, NCH, CHUNK)

    degp = _deg(dstb, zdeg, ones)
    sums1 = _agg(x, srcb, dstb, zrow)
    h1 = _tc_layer(x, sums1, degp, W_self1, W_neigh1, r1(b1),
                   g, be, mu, var)
    sums2 = _agg(h1, srcb, dstb, zrow)
    h2, p3 = _tc_layer2(h1, sums2, degp, W_self2, W_neigh2, r1(b2),
                        g, be, mu, var, wn3p)
    psums = _agg_cp(p3, srcb, dstb, zrow[:, :CP])
    return _tc_final(h2, psums, degp, W_self3, r1(b3))


# speedup vs baseline: 1.0213x; 1.0213x over previous
import jax
import jax.numpy as jnp
from jax import lax
from jax.experimental import pallas as pl
from jax.experimental.pallas import tpu as pltpu
from jax.experimental.pallas import tpu_sc as plsc

N = 10000
E = 320000
F = 128
NCLS = 40
CP = 48

NC, NS, LANES = 2, 16, 16
NW = NC * NS
EPW = E // NW
CHUNK = 100
NCH = EPW // CHUNK
NH = NCH // 2
RPW = N // NS

_mesh = plsc.VectorSubcoreMesh(core_axis_name="c", subcore_axis_name="s")
_sc_params = pltpu.CompilerParams(use_tc_tiling_on_sc=False)


def _make_agg(fw, chunk=CHUNK):
    nch = EPW // chunk
    nh = nch // 2

    def body(h_hbm, srcb_hbm, dstb_hbm, zrow_hbm, out_hbm,
             isrc, idst, rows0, rows1, acc, gsem0, gsem1, ssem0, ssem1):
        c = lax.axis_index("c")
        s = lax.axis_index("s")
        wid = s * NC + c
        r0 = s * RPW

        pltpu.sync_copy(srcb_hbm.at[wid], isrc)
        pltpu.sync_copy(dstb_hbm.at[wid], idst)
        pltpu.sync_copy(zrow_hbm.at[pl.ds(r0, RPW)], acc.at[pl.ds(r0, RPW)])
        plsc.subcore_barrier()

        pltpu.async_copy(h_hbm.at[isrc.at[0]], rows0, gsem0)

        @pl.loop(0, nh)
        def step(j):
            i0 = 2 * j
            i1 = i0 + 1
            pltpu.make_async_copy(h_hbm.at[isrc.at[i0]], rows0, gsem0).wait()

            @pl.when(j > 0)
            def _():
                pltpu.make_async_copy(rows1, acc.at[idst.at[i0 - 1]],
                                      ssem1).wait()

            pltpu.async_copy(h_hbm.at[isrc.at[i1]], rows1, gsem1)
            pltpu.async_copy(rows0, acc.at[idst.at[i0]], ssem0, add=True)
            pltpu.make_async_copy(h_hbm.at[isrc.at[i1]], rows1, gsem1).wait()
            pltpu.make_async_copy(rows0, acc.at[idst.at[i0]], ssem0).wait()
            pltpu.async_copy(rows1, acc.at[idst.at[i1]], ssem1, add=True)

            @pl.when(j < nh - 1)
            def _():
                pltpu.async_copy(h_hbm.at[isrc.at[i0 + 2]], rows0, gsem0)

        pltpu.make_async_copy(rows1, acc.at[idst.at[nch - 1]], ssem1).wait()
        plsc.subcore_barrier()
        pltpu.sync_copy(acc.at[pl.ds(r0, RPW)],
                        out_hbm.at[c, pl.ds(r0, RPW)])

    return pl.kernel(
        body,
        out_type=jax.ShapeDtypeStruct((NC, N, fw), jnp.float32),
        mesh=_mesh,
        scratch_types=(
            pltpu.VMEM((nch, chunk), jnp.int32),
            pltpu.VMEM((nch, chunk), jnp.int32),
            pltpu.VMEM((chunk, fw), jnp.float32),
            pltpu.VMEM((chunk, fw), jnp.float32),
            pltpu.VMEM_SHARED((N, fw), jnp.float32),
            pltpu.SemaphoreType.DMA,
            pltpu.SemaphoreType.DMA,
            pltpu.SemaphoreType.DMA,
            pltpu.SemaphoreType.DMA,
        ),
        compiler_params=_sc_params,
    )


_agg = _make_agg(F)
CP_CHUNK = 125
_agg_cp = _make_agg(CP, CP_CHUNK)


def _deg_body(dstb_hbm, zdeg_hbm, ones_hbm, deg_hbm,
              idst, ones_v, dacc, ssem):
    c = lax.axis_index("c")
    s = lax.axis_index("s")
    wid = s * NC + c
    r0 = s * RPW

    pltpu.sync_copy(dstb_hbm.at[wid], idst)
    pltpu.sync_copy(ones_hbm, ones_v)
    pltpu.sync_copy(zdeg_hbm.at[pl.ds(r0, RPW)], dacc.at[pl.ds(r0, RPW)])
    plsc.subcore_barrier()

    @pl.loop(0, NCH)
    def step(i):
        pltpu.async_copy(ones_v, dacc.at[idst.at[i]], ssem, add=True)

        @pl.when(i >= 8)
        def _():
            pltpu.make_async_copy(ones_v, dacc.at[idst.at[0]], ssem).wait()

    for _ in range(8):
        pltpu.make_async_copy(ones_v, dacc.at[idst.at[0]], ssem).wait()
    plsc.subcore_barrier()
    pltpu.sync_copy(dacc.at[pl.ds(r0, RPW)],
                    deg_hbm.at[c, pl.ds(r0, RPW)])


_deg = pl.kernel(
    _deg_body,
    out_type=jax.ShapeDtypeStruct((NC, N, LANES), jnp.float32),
    mesh=_mesh,
    scratch_types=(
        pltpu.VMEM((NCH, CHUNK), jnp.int32),
        pltpu.VMEM((CHUNK, LANES), jnp.float32),
        pltpu.VMEM_SHARED((N, LANES), jnp.float32),
        pltpu.SemaphoreType.DMA,
    ),
    compiler_params=_sc_params,
)


_RB = 1000


def _dot_t(a, w):
    return lax.dot_general(a, w, (((1,), (1,)), ((), ())),
                           preferred_element_type=jnp.float32)


def _tc_layer_body(h_ref, sums_ref, deg_ref, ws_ref, wn_ref, b_ref,
                   g_ref, be_ref, mu_ref, var_ref, o_ref):
    h = h_ref[...]
    sums = sums_ref[0] + sums_ref[1]
    deg = deg_ref[0, :, 0:1] + deg_ref[1, :, 0:1]
    hn = sums / jnp.maximum(deg, 1.0)
    z = _dot_t(h, ws_ref[...]) + _dot_t(hn, wn_ref[...]) + b_ref[...]
    scale = g_ref[...] * lax.rsqrt(var_ref[...] + 1e-5)
    shift = be_ref[...] - mu_ref[...] * scale
    y = z * scale + shift
    o_ref[...] = jnp.where(y >= 0, y, 0.01 * y)


def _tc_layer2_body(h_ref, sums_ref, deg_ref, ws_ref, wn_ref, b_ref,
                    g_ref, be_ref, mu_ref, var_ref, wn3p_ref,
                    o_ref, p_ref):
    h = h_ref[...]
    sums = sums_ref[0] + sums_ref[1]
    deg = deg_ref[0, :, 0:1] + deg_ref[1, :, 0:1]
    hn = sums / jnp.maximum(deg, 1.0)
    z = _dot_t(h, ws_ref[...]) + _dot_t(hn, wn_ref[...]) + b_ref[...]
    scale = g_ref[...] * lax.rsqrt(var_ref[...] + 1e-5)
    shift = be_ref[...] - mu_ref[...] * scale
    y = z * scale + shift
    h2 = jnp.where(y >= 0, y, 0.01 * y)
    o_ref[...] = h2
    p_ref[...] = _dot_t(h2, wn3p_ref[...])


def _tc_final_body(h_ref, psums_ref, deg_ref, ws3_ref, b3_ref, o_ref):
    h = h_ref[...]
    psums = psums_ref[0] + psums_ref[1]
    deg = deg_ref[0, :, 0:1] + deg_ref[1, :, 0:1]
    pn = psums / jnp.maximum(deg, 1.0)
    o_ref[...] = _dot_t(h, ws3_ref[...]) + b3_ref[...] + pn[:, :NCLS]


def _row_spec(w):
    return pl.BlockSpec((_RB, w), lambda i: (i, 0))


def _full_spec(shape):
    nd = len(shape)
    return pl.BlockSpec(shape, lambda i, _n=nd: (0,) * _n)


def _sums_spec(w):
    return pl.BlockSpec((NC, _RB, w), lambda i: (0, i, 0))


_GRID = N // _RB

_tc_layer = pl.pallas_call(
    _tc_layer_body,
    grid=(_GRID,),
    in_specs=[_row_spec(F), _sums_spec(F), _sums_spec(LANES),
              _full_spec((F, F)), _full_spec((F, F)), _full_spec((1, F)),
              _full_spec((1, F)), _full_spec((1, F)), _full_spec((1, F)),
              _full_spec((1, F))],
    out_specs=_row_spec(F),
    out_shape=jax.ShapeDtypeStruct((N, F), jnp.float32),
)

_tc_layer2 = pl.pallas_call(
    _tc_layer2_body,
    grid=(_GRID,),
    in_specs=[_row_spec(F), _sums_spec(F), _sums_spec(LANES),
              _full_spec((F, F)), _full_spec((F, F)), _full_spec((1, F)),
              _full_spec((1, F)), _full_spec((1, F)), _full_spec((1, F)),
              _full_spec((1, F)), _full_spec((CP, F))],
    out_specs=[_row_spec(F), _row_spec(CP)],
    out_shape=[jax.ShapeDtypeStruct((N, F), jnp.float32),
               jax.ShapeDtypeStruct((N, CP), jnp.float32)],
)

_tc_final = pl.pallas_call(
    _tc_final_body,
    grid=(_GRID,),
    in_specs=[_row_spec(F), _sums_spec(CP), _sums_spec(LANES),
              _full_spec((NCLS, F)), _full_spec((1, NCLS))],
    out_specs=_row_spec(NCLS),
    out_shape=jax.ShapeDtypeStruct((N, NCLS), jnp.float32),
)


def kernel(x, edge_index, W_self1, W_neigh1, b1, W_self2, W_neigh2, b2,
           W_self3, W_neigh3, b3, bn_gamma, bn_beta, bn_mean, bn_var):
    zrow = jnp.zeros((N, F), jnp.float32)
    zdeg = jnp.zeros((N, LANES), jnp.float32)
    ones = jnp.ones((CHUNK, LANES), jnp.float32)

    r1 = lambda v: v.reshape(1, -1)
    g, be, mu, var = r1(bn_gamma), r1(bn_beta), r1(bn_mean), r1(bn_var)

    wn3p = jnp.pad(W_neigh3, ((0, CP - W_neigh3.shape[0]), (0, 0)))

    srcb = edge_index[0].reshape(NW, NCH, CHUNK)
    dstb = edge_index[1].reshape(NW, NCH, CHUNK)

    degp = _deg(dstb, zdeg, ones)
    sums1 = _agg(x, srcb, dstb, zrow)
    h1 = _tc_layer(x, sums1, degp, W_self1, W_neigh1, r1(b1),
                   g, be, mu, var)
    sums2 = _agg(h1, srcb, dstb, zrow)
    h2, p3 = _tc_layer2(h1, sums2, degp, W_self2, W_neigh2, r1(b2),
                        g, be, mu, var, wn3p)
    srcb2 = edge_index[0].reshape(NW, EPW // CP_CHUNK, CP_CHUNK)
    dstb2 = edge_index[1].reshape(NW, EPW // CP_CHUNK, CP_CHUNK)
    psums = _agg_cp(p3, srcb2, dstb2, zrow[:, :CP])
    return _tc_final(h2, psums, degp, W_self3, r1(b3))
